# baseline (speedup 1.0000x reference)
import jax
import jax.numpy as jnp
from jax import lax
from jax.experimental import pallas as pl
from jax.experimental.pallas import tpu as pltpu
from jax.experimental.pallas import tpu_sc as plsc

N = 10000
E = 320000
D = 128
NP = 10112
P = 3
C = 32

XS = 80
X1 = D - XS
W0 = 80
W1R = 64
GB = X1

NS = 16
K = 128
ER = 2512
NCH = ER // NS
RPT = NP // NS


def _sc_body(eij_hbm, t0_hbm, t1_hbm, pp_hbm, z0_hbm, z1_hbm,
             s0_out, s1_out,
             eij3, rb0, rb1, pbi, s0_sh, s1_sh, gsem, psem, isem, ssem):
    c = lax.axis_index("c")
    s = lax.axis_index("s")
    lanes = lax.iota(jnp.int32, 16)
    r0 = s * RPT

    for j in range(4):
        @pl.when(c == 0)
        def _():
            pltpu.sync_copy(z0_hbm, s0_sh.at[pl.ds(r0 + j * 128, 128)])

        @pl.when(c == 1)
        def _():
            pltpu.sync_copy(z1_hbm, s1_sh.at[pl.ds(r0 + j * 128, 128)])

    @pl.when(c == 0)
    def _():
        pltpu.sync_copy(z0_hbm.at[pl.ds(0, RPT - 512)],
                        s0_sh.at[pl.ds(r0 + 512, RPT - 512)])

    @pl.when(c == 1)
    def _():
        pltpu.sync_copy(z1_hbm.at[pl.ds(0, RPT - 512)],
                        s1_sh.at[pl.ds(r0 + 512, RPT - 512)])

    plsc.subcore_barrier()

    def issue_idx(m, slot):
        r = s + 16 * m
        pltpu.async_copy(eij_hbm.at[pl.ds(r, 1)], eij3.at[pl.ds(slot, 1)],
                         isem)

    def wait_idx():
        pltpu.make_async_copy(
            eij_hbm.at[pl.ds(0, 1)], eij3.at[pl.ds(0, 1)], isem).wait()

    def issue_gather(b, slot):
        @pl.when(c == 0)
        def _():
            pltpu.async_copy(t0_hbm.at[eij3.at[slot, 1]], rb0.at[b], gsem)

        @pl.when(c == 1)
        def _():
            pltpu.async_copy(t1_hbm.at[eij3.at[slot, 1]], rb1.at[b], gsem)
            pltpu.async_copy(pp_hbm.at[eij3.at[slot, 0]], pbi, psem)

    def wait_gather():
        @pl.when(c == 0)
        def _():
            pltpu.make_async_copy(
                t0_hbm.at[eij3.at[0, 1]], rb0.at[0], gsem).wait()

        @pl.when(c == 1)
        def _():
            pltpu.make_async_copy(
                t1_hbm.at[eij3.at[0, 1]], rb1.at[0], gsem).wait()
            pltpu.make_async_copy(
                pp_hbm.at[eij3.at[0, 0]], pbi, psem).wait()

    def issue_scatter(b, slot):
        @pl.when(c == 0)
        def _():
            pltpu.async_copy(rb0.at[b], s0_sh.at[eij3.at[slot, 0]], ssem,
                             add=True)

        @pl.when(c == 1)
        def _():
            pltpu.async_copy(rb1.at[b], s1_sh.at[eij3.at[slot, 0]], ssem,
                             add=True)

    def wait_scatter():
        @pl.when(c == 0)
        def _():
            pltpu.make_async_copy(
                rb0.at[0], s0_sh.at[eij3.at[0, 0]], ssem).wait()

        @pl.when(c == 1)
        def _():
            pltpu.make_async_copy(
                rb1.at[0], s1_sh.at[eij3.at[0, 0]], ssem).wait()

    pltpu.sync_copy(eij_hbm.at[pl.ds(s, 1)], eij3.at[pl.ds(0, 1)])
    issue_gather(0, 0)
    issue_idx(1, 1)

    def step(m, carry):
        b = lax.rem(m, 2)
        nb = 1 - b
        s_cur = lax.rem(m, 3)
        s_nxt = lax.rem(m + 1, 3)
        s_new = lax.rem(m + 2, 3)

        wait_gather()

        @pl.when(c == 1)
        def _():
            bvec = jnp.full((16,), 0, jnp.int32) + b
            for tt in range(K // 16):
                rows = tt * 16 + lanes
                col0 = jnp.zeros((16,), jnp.int32)
                pix = plsc.load_gather(pbi, [rows, col0])
                piy = plsc.load_gather(pbi, [rows, col0 + 1])
                piz = plsc.load_gather(pbi, [rows, col0 + 2])
                pjx = plsc.load_gather(rb1, [bvec, rows, col0 + GB])
                pjy = plsc.load_gather(rb1, [bvec, rows, col0 + GB + 1])
                pjz = plsc.load_gather(rb1, [bvec, rows, col0 + GB + 2])
                dx = pjx - pix
                dy = pjy - piy
                dz = pjz - piz
                d2 = dx * dx + dy * dy + dz * dz
                yi = jnp.int32(0x5F3759DF) - lax.shift_right_logical(
                    plsc.bitcast(d2, jnp.int32), 1)
                y = plsc.bitcast(yi, jnp.float32)
                h = d2 * 0.5
                for _ in range(3):
                    y = y * (1.5 - h * y * y)
                one = jnp.full((16,), 1.0, jnp.float32)
                plsc.store_scatter(rb1, [bvec, rows, col0 + GB], one)
                plsc.store_scatter(rb1, [bvec, rows, col0 + GB + 1], d2 * y)
                plsc.store_scatter(rb1, [bvec, rows, col0 + GB + 2], dx * y)
                plsc.store_scatter(rb1, [bvec, rows, col0 + GB + 3], dy * y)
                plsc.store_scatter(rb1, [bvec, rows, col0 + GB + 4], dz * y)

        issue_scatter(b, s_cur)

        @pl.when(m > 0)
        def _():
            wait_scatter()

        @pl.when(m < NCH - 2)
        def _():
            issue_idx(m + 2, s_new)

        @pl.when(m < NCH - 1)
        def _():
            wait_idx()
            issue_gather(nb, s_nxt)

        return carry

    lax.fori_loop(0, NCH, step, 0)

    wait_scatter()
    plsc.subcore_barrier()

    @pl.when(c == 0)
    def _():
        pltpu.sync_copy(s0_sh.at[pl.ds(r0, RPT)], s0_out.at[pl.ds(r0, RPT)])

    @pl.when(c == 1)
    def _():
        pltpu.sync_copy(s1_sh.at[pl.ds(r0, RPT)], s1_out.at[pl.ds(r0, RPT)])


def _sc_segment_sums(eij, t0, t1, pp, z0, z1):
    mesh = plsc.VectorSubcoreMesh(core_axis_name="c", subcore_axis_name="s")
    return pl.kernel(
        _sc_body,
        out_type=(
            jax.ShapeDtypeStruct((NP, W0), jnp.float32),
            jax.ShapeDtypeStruct((NP, W1R), jnp.float32),
        ),
        mesh=mesh,
        compiler_params=pltpu.CompilerParams(
            needs_layout_passes=False, use_tc_tiling_on_sc=False),
        scratch_types=[
            pltpu.VMEM((3, 2, K), jnp.int32),
            pltpu.VMEM((2, K, W0), jnp.float32),
            pltpu.VMEM((2, K, W1R), jnp.float32),
            pltpu.VMEM((K, 16), jnp.float32),
            pltpu.VMEM_SHARED((NP, W0), jnp.float32),
            pltpu.VMEM_SHARED((NP, W1R), jnp.float32),
            pltpu.SemaphoreType.DMA,
            pltpu.SemaphoreType.DMA,
            pltpu.SemaphoreType.DMA,
            pltpu.SemaphoreType.DMA,
        ],
    )(eij, t0, t1, pp, z0, z1)


def _tc_body(s0_ref, s1_ref, x_ref, w1p_ref, b1_ref, a1_ref, ba1_ref,
             w2a_ref, w2b0_ref, w2b1_ref, w2cp_ref, w2d_ref, b2_ref,
             a2a_ref, a2b_ref, ba2_ref, out_ref):
    f32 = jnp.float32
    s0 = s0_ref[...]
    s1 = s1_ref[...]
    g = s1[:, GB:GB + 16]
    cnt = g[:, :1]
    inv = 1.0 / jnp.maximum(cnt, 1.0)
    has = jnp.minimum(cnt, 1.0)
    aggr1 = jnp.dot(g, w1p_ref[...], preferred_element_type=f32) * inv \
        + has * b1_ref[...]
    ctx = jnp.dot(aggr1, a1_ref[...], preferred_element_type=f32) + ba1_ref[...]
    t = jnp.dot(x_ref[...], w2a_ref[...], preferred_element_type=f32) \
        + jnp.dot(ctx, w2d_ref[...], preferred_element_type=f32) + b2_ref[...]
    sterm = jnp.dot(s0, w2b0_ref[...], preferred_element_type=f32) \
        + jnp.dot(s1[:, :X1], w2b1_ref[...], preferred_element_type=f32)
    aggr2 = (cnt * t + sterm
             + jnp.dot(g, w2cp_ref[...], preferred_element_type=f32)) * inv
    out_ref[...] = jnp.dot(aggr2, a2a_ref[...], preferred_element_type=f32) \
        + jnp.dot(ctx, a2b_ref[...], preferred_element_type=f32) + ba2_ref[...]


def _tc_combine(s0, s1, x, w1p, b1, a1, ba1, wx, w2b0, w2b1, w2cp, w2d, b2,
                a2a, a2b, ba2):
    bn = 1000
    grid = (N // bn,)
    full = lambda shape: pl.BlockSpec(shape, lambda i: (0,) * len(shape))
    return pl.pallas_call(
        _tc_body,
        grid=grid,
        in_specs=[
            pl.BlockSpec((bn, W0), lambda i: (i, 0)),
            pl.BlockSpec((bn, W1R), lambda i: (i, 0)),
            pl.BlockSpec((bn, D), lambda i: (i, 0)),
            full((16, C)), full((1, C)), full((C, C)), full((1, C)),
            full((D, D)), full((XS, D)), full((X1, D)), full((16, D)),
            full((C, D)), full((1, D)), full((D, D)), full((C, D)),
            full((1, D)),
        ],
        out_specs=pl.BlockSpec((bn, D), lambda i: (i, 0)),
        out_shape=jax.ShapeDtypeStruct((N, D), jnp.float32),
    )(s0, s1, x, w1p, b1, a1, ba1, wx, w2b0, w2b1, w2cp, w2d, b2,
      a2a, a2b, ba2)


def kernel(x, edge_index, pos, W1, b1, A1, bA1, W2, b2, A2, bA2):
    ei = edge_index[0]
    ej = edge_index[1]
    npad = ER * K - E
    eip = jnp.concatenate([ei, jnp.full((npad,), NP - 1, jnp.int32)])
    ejp = jnp.concatenate([ej, jnp.zeros((npad,), jnp.int32)])
    eij = jnp.stack([eip.reshape(ER, K), ejp.reshape(ER, K)], axis=1)
    t0 = x[:, :XS]
    t1 = jnp.concatenate(
        [x[:, XS:], pos, jnp.zeros((N, W1R - X1 - P), jnp.float32)], axis=1)
    pp = jnp.zeros((NP, 16), jnp.float32).at[:N, :P].set(pos)
    z0 = jnp.zeros((128, W0), jnp.float32)
    z1 = jnp.zeros((128, W1R), jnp.float32)

    s0, s1 = _sc_segment_sums(eij, t0, t1, pp, z0, z1)

    w1p = jnp.zeros((16, C), jnp.float32).at[1:1 + P + 1].set(W1)
    wx = W2[:D] - W2[D:2 * D]
    w2b0 = W2[D:D + XS]
    w2b1 = W2[D + XS:2 * D]
    w2cp = jnp.zeros((16, D), jnp.float32).at[1:1 + P + 1].set(
        W2[2 * D:2 * D + P + 1])
    w2d = W2[2 * D + P + 1:]
    a2a = A2[:D]
    a2b = A2[D:]

    return _tc_combine(
        s0, s1, x,
        w1p, b1.reshape(1, C), A1, bA1.reshape(1, C),
        wx, w2b0, w2b1, w2cp, w2d, b2.reshape(1, D),
        a2a, a2b, bA2.reshape(1, D))


# speedup vs baseline: 1.0783x; 1.0783x over previous
import jax
import jax.numpy as jnp
from jax import lax
from jax.experimental import pallas as pl
from jax.experimental.pallas import tpu as pltpu
from jax.experimental.pallas import tpu_sc as plsc

N = 10000
E = 320000
D = 128
NP = 10112
P = 3
C = 32

XS = 80
X1 = D - XS
W0 = 80
W1R = 64
GB = X1

NS = 16
K = 128
ER = 2512
NCH = ER // NS
RPT = NP // NS


def _sc_body(eij_hbm, t0_hbm, t1_hbm, pp_hbm, z0_hbm, z1_hbm,
             s0_out, s1_out,
             eij3, rb0, rb1, pbi, s0_sh, s1_sh, gsem, psem, isem, ssem):
    c = lax.axis_index("c")
    s = lax.axis_index("s")
    lanes = lax.iota(jnp.int32, 16)
    r0 = s * RPT

    for j in range(4):
        @pl.when(c == 0)
        def _():
            pltpu.sync_copy(z0_hbm, s0_sh.at[pl.ds(r0 + j * 128, 128)])

        @pl.when(c == 1)
        def _():
            pltpu.sync_copy(z1_hbm, s1_sh.at[pl.ds(r0 + j * 128, 128)])

    @pl.when(c == 0)
    def _():
        pltpu.sync_copy(z0_hbm.at[pl.ds(0, RPT - 512)],
                        s0_sh.at[pl.ds(r0 + 512, RPT - 512)])

    @pl.when(c == 1)
    def _():
        pltpu.sync_copy(z1_hbm.at[pl.ds(0, RPT - 512)],
                        s1_sh.at[pl.ds(r0 + 512, RPT - 512)])

    plsc.subcore_barrier()

    def issue_idx(m, slot):
        r = s + 16 * m
        pltpu.async_copy(eij_hbm.at[pl.ds(r, 1)], eij3.at[pl.ds(slot, 1)],
                         isem)

    def wait_idx():
        pltpu.make_async_copy(
            eij_hbm.at[pl.ds(0, 1)], eij3.at[pl.ds(0, 1)], isem).wait()

    def issue_rows(b, slot):
        @pl.when(c == 0)
        def _():
            pltpu.async_copy(t0_hbm.at[eij3.at[slot, 1]], rb0.at[b], gsem)

        @pl.when(c == 1)
        def _():
            pltpu.async_copy(t1_hbm.at[eij3.at[slot, 1]], rb1.at[b], gsem)

    def issue_pbi(slot):
        @pl.when(c == 1)
        def _():
            pltpu.async_copy(pp_hbm.at[eij3.at[slot, 0]], pbi, psem)

    def issue_gather(b, slot):
        issue_rows(b, slot)
        issue_pbi(slot)

    def wait_gather():
        @pl.when(c == 0)
        def _():
            pltpu.make_async_copy(
                t0_hbm.at[eij3.at[0, 1]], rb0.at[0], gsem).wait()

        @pl.when(c == 1)
        def _():
            pltpu.make_async_copy(
                t1_hbm.at[eij3.at[0, 1]], rb1.at[0], gsem).wait()
            pltpu.make_async_copy(
                pp_hbm.at[eij3.at[0, 0]], pbi, psem).wait()

    def issue_scatter(b, slot):
        @pl.when(c == 0)
        def _():
            pltpu.async_copy(rb0.at[b], s0_sh.at[eij3.at[slot, 0]], ssem,
                             add=True)

        @pl.when(c == 1)
        def _():
            pltpu.async_copy(rb1.at[b], s1_sh.at[eij3.at[slot, 0]], ssem,
                             add=True)

    def wait_scatter():
        @pl.when(c == 0)
        def _():
            pltpu.make_async_copy(
                rb0.at[0], s0_sh.at[eij3.at[0, 0]], ssem).wait()

        @pl.when(c == 1)
        def _():
            pltpu.make_async_copy(
                rb1.at[0], s1_sh.at[eij3.at[0, 0]], ssem).wait()

    pltpu.sync_copy(eij_hbm.at[pl.ds(s, 1)], eij3.at[pl.ds(0, 1)])
    issue_gather(0, 0)
    issue_idx(1, 1)

    def step(m, carry):
        b = lax.rem(m, 2)
        nb = 1 - b
        s_cur = lax.rem(m, 3)
        s_nxt = lax.rem(m + 1, 3)
        s_new = lax.rem(m + 2, 3)

        wait_gather()

        @pl.when(m > 0)
        def _():
            wait_scatter()

        @pl.when(m < NCH - 1)
        def _():
            wait_idx()
            issue_rows(nb, s_nxt)

        @pl.when(m < NCH - 2)
        def _():
            issue_idx(m + 2, s_new)

        @pl.when(c == 1)
        def _():
            bvec = jnp.full((16,), 0, jnp.int32) + b
            for tt in range(K // 16):
                rows = tt * 16 + lanes
                col0 = jnp.zeros((16,), jnp.int32)
                pix = plsc.load_gather(pbi, [rows, col0])
                piy = plsc.load_gather(pbi, [rows, col0 + 1])
                piz = plsc.load_gather(pbi, [rows, col0 + 2])
                pjx = plsc.load_gather(rb1, [bvec, rows, col0 + GB])
                pjy = plsc.load_gather(rb1, [bvec, rows, col0 + GB + 1])
                pjz = plsc.load_gather(rb1, [bvec, rows, col0 + GB + 2])
                dx = pjx - pix
                dy = pjy - piy
                dz = pjz - piz
                d2 = dx * dx + dy * dy + dz * dz
                yi = jnp.int32(0x5F3759DF) - lax.shift_right_logical(
                    plsc.bitcast(d2, jnp.int32), 1)
                y = plsc.bitcast(yi, jnp.float32)
                h = d2 * 0.5
                for _ in range(3):
                    y = y * (1.5 - h * y * y)
                one = jnp.full((16,), 1.0, jnp.float32)
                plsc.store_scatter(rb1, [bvec, rows, col0 + GB], one)
                plsc.store_scatter(rb1, [bvec, rows, col0 + GB + 1], d2 * y)
                plsc.store_scatter(rb1, [bvec, rows, col0 + GB + 2], dx * y)
                plsc.store_scatter(rb1, [bvec, rows, col0 + GB + 3], dy * y)
                plsc.store_scatter(rb1, [bvec, rows, col0 + GB + 4], dz * y)

        @pl.when(m < NCH - 1)
        def _():
            issue_pbi(s_nxt)

        issue_scatter(b, s_cur)

        return carry

    lax.fori_loop(0, NCH, step, 0)

    wait_scatter()
    plsc.subcore_barrier()

    @pl.when(c == 0)
    def _():
        pltpu.sync_copy(s0_sh.at[pl.ds(r0, RPT)], s0_out.at[pl.ds(r0, RPT)])

    @pl.when(c == 1)
    def _():
        pltpu.sync_copy(s1_sh.at[pl.ds(r0, RPT)], s1_out.at[pl.ds(r0, RPT)])


def _sc_segment_sums(eij, t0, t1, pp, z0, z1):
    mesh = plsc.VectorSubcoreMesh(core_axis_name="c", subcore_axis_name="s")
    return pl.kernel(
        _sc_body,
        out_type=(
            jax.ShapeDtypeStruct((NP, W0), jnp.float32),
            jax.ShapeDtypeStruct((NP, W1R), jnp.float32),
        ),
        mesh=mesh,
        compiler_params=pltpu.CompilerParams(
            needs_layout_passes=False, use_tc_tiling_on_sc=False),
        scratch_types=[
            pltpu.VMEM((3, 2, K), jnp.int32),
            pltpu.VMEM((2, K, W0), jnp.float32),
            pltpu.VMEM((2, K, W1R), jnp.float32),
            pltpu.VMEM((K, 16), jnp.float32),
            pltpu.VMEM_SHARED((NP, W0), jnp.float32),
            pltpu.VMEM_SHARED((NP, W1R), jnp.float32),
            pltpu.SemaphoreType.DMA,
            pltpu.SemaphoreType.DMA,
            pltpu.SemaphoreType.DMA,
            pltpu.SemaphoreType.DMA,
        ],
    )(eij, t0, t1, pp, z0, z1)


def _tc_body(s0_ref, s1_ref, x_ref, w1p_ref, b1_ref, a1_ref, ba1_ref,
             w2a_ref, w2b0_ref, w2b1_ref, w2cp_ref, w2d_ref, b2_ref,
             a2a_ref, a2b_ref, ba2_ref, out_ref):
    f32 = jnp.float32
    s0 = s0_ref[...]
    s1 = s1_ref[...]
    g = s1[:, GB:GB + 16]
    cnt = g[:, :1]
    inv = 1.0 / jnp.maximum(cnt, 1.0)
    has = jnp.minimum(cnt, 1.0)
    aggr1 = jnp.dot(g, w1p_ref[...], preferred_element_type=f32) * inv \
        + has * b1_ref[...]
    ctx = jnp.dot(aggr1, a1_ref[...], preferred_element_type=f32) + ba1_ref[...]
    t = jnp.dot(x_ref[...], w2a_ref[...], preferred_element_type=f32) \
        + jnp.dot(ctx, w2d_ref[...], preferred_element_type=f32) + b2_ref[...]
    sterm = jnp.dot(s0, w2b0_ref[...], preferred_element_type=f32) \
        + jnp.dot(s1[:, :X1], w2b1_ref[...], preferred_element_type=f32)
    aggr2 = (cnt * t + sterm
             + jnp.dot(g, w2cp_ref[...], preferred_element_type=f32)) * inv
    out_ref[...] = jnp.dot(aggr2, a2a_ref[...], preferred_element_type=f32) \
        + jnp.dot(ctx, a2b_ref[...], preferred_element_type=f32) + ba2_ref[...]


def _tc_combine(s0, s1, x, w1p, b1, a1, ba1, wx, w2b0, w2b1, w2cp, w2d, b2,
                a2a, a2b, ba2):
    bn = 1000
    grid = (N // bn,)
    full = lambda shape: pl.BlockSpec(shape, lambda i: (0,) * len(shape))
    return pl.pallas_call(
        _tc_body,
        grid=grid,
        in_specs=[
            pl.BlockSpec((bn, W0), lambda i: (i, 0)),
            pl.BlockSpec((bn, W1R), lambda i: (i, 0)),
            pl.BlockSpec((bn, D), lambda i: (i, 0)),
            full((16, C)), full((1, C)), full((C, C)), full((1, C)),
            full((D, D)), full((XS, D)), full((X1, D)), full((16, D)),
            full((C, D)), full((1, D)), full((D, D)), full((C, D)),
            full((1, D)),
        ],
        out_specs=pl.BlockSpec((bn, D), lambda i: (i, 0)),
        out_shape=jax.ShapeDtypeStruct((N, D), jnp.float32),
    )(s0, s1, x, w1p, b1, a1, ba1, wx, w2b0, w2b1, w2cp, w2d, b2,
      a2a, a2b, ba2)


def kernel(x, edge_index, pos, W1, b1, A1, bA1, W2, b2, A2, bA2):
    ei = edge_index[0]
    ej = edge_index[1]
    npad = ER * K - E
    eip = jnp.concatenate([ei, jnp.full((npad,), NP - 1, jnp.int32)])
    ejp = jnp.concatenate([ej, jnp.zeros((npad,), jnp.int32)])
    eij = jnp.stack([eip.reshape(ER, K), ejp.reshape(ER, K)], axis=1)
    t0 = x[:, :XS]
    t1 = jnp.concatenate(
        [x[:, XS:], pos, jnp.zeros((N, W1R - X1 - P), jnp.float32)], axis=1)
    pp = jnp.zeros((NP, 16), jnp.float32).at[:N, :P].set(pos)
    z0 = jnp.zeros((128, W0), jnp.float32)
    z1 = jnp.zeros((128, W1R), jnp.float32)

    s0, s1 = _sc_segment_sums(eij, t0, t1, pp, z0, z1)

    w1p = jnp.zeros((16, C), jnp.float32).at[1:1 + P + 1].set(W1)
    wx = W2[:D] - W2[D:2 * D]
    w2b0 = W2[D:D + XS]
    w2b1 = W2[D + XS:2 * D]
    w2cp = jnp.zeros((16, D), jnp.float32).at[1:1 + P + 1].set(
        W2[2 * D:2 * D + P + 1])
    w2d = W2[2 * D + P + 1:]
    a2a = A2[:D]
    a2b = A2[D:]

    return _tc_combine(
        s0, s1, x,
        w1p, b1.reshape(1, C), A1, bA1.reshape(1, C),
        wx, w2b0, w2b1, w2cp, w2d, b2.reshape(1, D),
        a2a, a2b, bA2.reshape(1, D))
